# Initial kernel scaffold; baseline (speedup 1.0000x reference)
import jax
import jax.numpy as jnp
from jax.experimental import pallas as pl


def kernel(X_list, edge_index_list, W_list, Y_hist_list, hidden_in, edge_weight_list, W_phi, b_phi, gc_W, gc_b, W_fuse, b_fuse, Wih, bih, Whh, bhh, Wd, bd, dw1, db1, dw2, db2):
    raise NotImplementedError("write your pallas kernel here")


# speedup vs baseline: 12.4090x; 12.4090x over previous
import functools

import jax
import jax.numpy as jnp
from jax import lax
from jax.experimental import pallas as pl
from jax.experimental.pallas import tpu as pltpu
from jax.experimental.pallas import tpu_sc as plsc

N = 10000
E = 320000
TT = 4
D = 128
GRID = 10
NB = 1000
NBLK = N // NB
NWORK = 32
NP = 10240


def _phi_xw_body(x_ref, wphi_ref, bphi_ref, gcw_ref, phi_ref, xw_ref):
    x = x_ref[0]
    phi = jnp.maximum(
        jnp.dot(x, wphi_ref[...], preferred_element_type=jnp.float32)
        + bphi_ref[...], 0.0)
    phi_ref[0] = phi
    xw_ref[0] = jnp.dot(phi, gcw_ref[0], preferred_element_type=jnp.float32)


def _phi_xw(X, W_phi, b_phi2, gc_W):
    return pl.pallas_call(
        _phi_xw_body,
        grid=(TT, NBLK),
        in_specs=[
            pl.BlockSpec((1, NB, D), lambda t, b: (t, b, 0)),
            pl.BlockSpec((D, D), lambda t, b: (0, 0)),
            pl.BlockSpec((1, D), lambda t, b: (0, 0)),
            pl.BlockSpec((1, D, D), lambda t, b: (t, 0, 0)),
        ],
        out_specs=[
            pl.BlockSpec((1, NB, D), lambda t, b: (t, b, 0)),
            pl.BlockSpec((1, NB, D), lambda t, b: (t, b, 0)),
        ],
        out_shape=[
            jax.ShapeDtypeStruct((TT, N, D), jnp.float32),
            jax.ShapeDtypeStruct((TT, N, D), jnp.float32),
        ],
    )(X, W_phi, b_phi2, gc_W)


def _dinv_body(dp_ref, dinv_ref, dd_ref):
    deg = jnp.sum(dp_ref[0], axis=0, keepdims=True) + 1.0
    dinv = lax.rsqrt(deg)
    dinv_ref[0] = dinv
    dd_ref[0] = dinv * dinv


def _dinv(deg_parts):
    return pl.pallas_call(
        _dinv_body,
        grid=(TT,),
        in_specs=[pl.BlockSpec((1, NWORK, N), lambda t: (t, 0, 0))],
        out_specs=[
            pl.BlockSpec((1, 1, N), lambda t: (t, 0, 0)),
            pl.BlockSpec((1, 1, N), lambda t: (t, 0, 0)),
        ],
        out_shape=[
            jax.ShapeDtypeStruct((TT, 1, N), jnp.float32),
            jax.ShapeDtypeStruct((TT, 1, N), jnp.float32),
        ],
    )(deg_parts)


def _stage2_body(phi_ref, xw_ref, agg_ref, dinv_ref, dd_ref,
                 h_ref, w_ref, yh_ref, gcb_ref, wfp_ref, wfh_ref, wfr_ref,
                 bf_ref, wd_ref, bd_ref, dw1_ref, db1_ref, dw2_ref, db2_ref,
                 wihz_ref, wihw_ref, wihy_ref, bih_ref, whh_ref, bhh_ref,
                 y_ref, g_ref, hout_ref):
    f32 = jnp.float32
    phi = phi_ref[0]
    xwv = xw_ref[0]
    h = h_ref[...]
    agg = agg_ref[0, 0] + agg_ref[0, 1]
    rep = jnp.maximum(
        dinv_ref[0] * agg + dd_ref[0] * xwv + gcb_ref[0], 0.0)
    z = jnp.maximum(
        jnp.dot(phi, wfp_ref[...], preferred_element_type=f32)
        + jnp.dot(h, wfh_ref[...], preferred_element_type=f32)
        + jnp.dot(rep, wfr_ref[...], preferred_element_type=f32)
        + bf_ref[...], 0.0)

    logits = jnp.dot(z, wd_ref[...], preferred_element_type=f32) + bd_ref[...]
    m = jnp.max(logits, axis=1, keepdims=True)
    ex = jnp.exp(logits - m)
    sm = ex / jnp.sum(ex, axis=1, keepdims=True)
    treat = w_ref[0]
    tg = treat * float(GRID)
    U = jnp.ceil(tg)
    inter = 1.0 - (U - tg)
    L = U - 1.0
    L = L + (L < 0).astype(f32)
    Li = jnp.clip(L.astype(jnp.int32), 0, GRID)
    Ui = jnp.clip(U.astype(jnp.int32), 0, GRID)
    cols = lax.broadcasted_iota(jnp.int32, (NB, D), 1)
    Lout = jnp.sum(jnp.where(cols == Li, sm, 0.0), axis=1, keepdims=True)
    Uout = jnp.sum(jnp.where(cols == Ui, sm, 0.0), axis=1, keepdims=True)
    g_ref[...] = Lout + (Uout - Lout) * inter

    b0 = jnp.ones_like(treat)
    b1 = treat
    b2 = treat * treat
    b3 = jnp.maximum(treat - 0.33, 0.0) ** 2
    b4 = jnp.maximum(treat - 0.66, 0.0) ** 2
    basis5 = jnp.concatenate([b0, b1, b2, b3, b4], axis=1)
    o1 = (b0 * jnp.dot(z, dw1_ref[0], preferred_element_type=f32)
          + b1 * jnp.dot(z, dw1_ref[1], preferred_element_type=f32)
          + b2 * jnp.dot(z, dw1_ref[2], preferred_element_type=f32)
          + b3 * jnp.dot(z, dw1_ref[3], preferred_element_type=f32)
          + b4 * jnp.dot(z, dw1_ref[4], preferred_element_type=f32))
    o1 = o1 + jnp.dot(basis5, db1_ref[...], preferred_element_type=f32)
    o1 = jnp.maximum(o1, 0.0)
    y5 = jnp.dot(o1, dw2_ref[...], preferred_element_type=f32)
    y = (jnp.sum(basis5 * y5, axis=1, keepdims=True)
         + jnp.sum(basis5 * db2_ref[...], axis=1, keepdims=True))
    y_ref[...] = y

    gi = (jnp.dot(z, wihz_ref[...], preferred_element_type=f32)
          + treat * wihw_ref[...] + yh_ref[0] * wihy_ref[...]
          + bih_ref[...])
    gh = jnp.dot(h, whh_ref[...], preferred_element_type=f32) + bhh_ref[...]
    r = jax.nn.sigmoid(gi[:, :D] + gh[:, :D])
    zg = jax.nn.sigmoid(gi[:, D:2 * D] + gh[:, D:2 * D])
    n = jnp.tanh(gi[:, 2 * D:] + r * gh[:, 2 * D:])
    hout_ref[...] = (1.0 - zg) * n + zg * h


def _stage2(t, phi, xw, aggp, dinv_col, dd_col, h, W_list, Y_hist, gcb3,
            wfp, wfh, wfr, bf, wd, bd, dw1m, db1t, dw2f, db2r, wihz, wihw,
            wihy, bih2, whht, bhh2):
    tb = lambda b: (t, b, 0)
    nodeb = lambda b: (b, 0)
    full = lambda b: (0, 0)
    full3 = lambda b: (0, 0, 0)
    return pl.pallas_call(
        _stage2_body,
        grid=(NBLK,),
        in_specs=[
            pl.BlockSpec((1, NB, D), tb),
            pl.BlockSpec((1, NB, D), tb),
            pl.BlockSpec((1, 2, NB, D), lambda b: (t, 0, b, 0)),
            pl.BlockSpec((1, NB, 1), tb),
            pl.BlockSpec((1, NB, 1), tb),
            pl.BlockSpec((NB, D), nodeb),
            pl.BlockSpec((1, NB, 1), tb),
            pl.BlockSpec((1, NB, 1), tb),
            pl.BlockSpec((1, 1, D), lambda b: (t, 0, 0)),
            pl.BlockSpec((D, D), full),
            pl.BlockSpec((D, D), full),
            pl.BlockSpec((D, D), full),
            pl.BlockSpec((1, D), full),
            pl.BlockSpec((D, D), full),
            pl.BlockSpec((1, D), full),
            pl.BlockSpec((5, D, 64), full3),
            pl.BlockSpec((5, 64), full),
            pl.BlockSpec((64, 5), full),
            pl.BlockSpec((1, 5), full),
            pl.BlockSpec((D, 3 * D), full),
            pl.BlockSpec((1, 3 * D), full),
            pl.BlockSpec((1, 3 * D), full),
            pl.BlockSpec((1, 3 * D), full),
            pl.BlockSpec((D, 3 * D), full),
            pl.BlockSpec((1, 3 * D), full),
        ],
        out_specs=[
            pl.BlockSpec((NB, 1), nodeb),
            pl.BlockSpec((NB, 1), nodeb),
            pl.BlockSpec((NB, D), nodeb),
        ],
        out_shape=[
            jax.ShapeDtypeStruct((N, 1), jnp.float32),
            jax.ShapeDtypeStruct((N, 1), jnp.float32),
            jax.ShapeDtypeStruct((N, D), jnp.float32),
        ],
    )(phi, xw, aggp, dinv_col, dd_col, h, W_list, Y_hist, gcb3, wfp, wfh,
      wfr, bf, wd, bd, dw1m, db1t, dw2f, db2r, wihz, wihw, wihy, bih2,
      whht, bhh2)


EPW = E // NWORK
DEG_CH = 2000
AGG_CH = 80
RPTP = NP // 16
ZR = 160


@functools.cache
def _sc_kernels():
    mesh = plsc.VectorSubcoreMesh(core_axis_name="c", subcore_axis_name="s")
    params = pltpu.CompilerParams(needs_layout_passes=False)
    deg_k = functools.partial(
        pl.kernel,
        out_type=jax.ShapeDtypeStruct((TT * NWORK * N,), jnp.float32),
        mesh=mesh,
        compiler_params=params,
        scratch_types=[
            pltpu.VMEM((N,), jnp.float32),
            pltpu.VMEM((DEG_CH,), jnp.int32),
            pltpu.VMEM((DEG_CH,), jnp.float32),
        ],
    )(_sc_deg_body)
    agg_k = functools.partial(
        pl.kernel,
        out_type=jax.ShapeDtypeStruct((TT, 2, NP, D), jnp.float32),
        mesh=mesh,
        compiler_params=params,
        scratch_types=[
            pltpu.VMEM((N,), jnp.float32),
            pltpu.VMEM((AGG_CH,), jnp.int32),
            pltpu.VMEM((AGG_CH,), jnp.int32),
            pltpu.VMEM((AGG_CH,), jnp.float32),
            pltpu.VMEM((AGG_CH, D), jnp.float32),
            pltpu.VMEM((ZR, D), jnp.float32),
            pltpu.VMEM_SHARED((NP, D), jnp.float32),
            pltpu.SemaphoreType.DMA,
        ],
    )(_sc_agg_body)
    return deg_k, agg_k


def _sc_deg_body(dst_hbm, ew_hbm, out_hbm, deg_v, idx_v, ew_v):
    cid = lax.axis_index("c")
    sid = lax.axis_index("s")
    wid = sid * 2 + cid
    for t in range(TT):
        def zero_body(i, c):
            deg_v[pl.ds(i * 16, 16)] = jnp.zeros((16,), jnp.float32)
            return c
        lax.fori_loop(0, N // 16, zero_body, 0)
        base = t * E + wid * EPW
        for g in range(EPW // DEG_CH):
            sl = pl.ds(base + g * DEG_CH, DEG_CH)
            pltpu.sync_copy(dst_hbm.at[sl], idx_v)
            pltpu.sync_copy(ew_hbm.at[sl], ew_v)

            def acc_body(j, c):
                ii = idx_v[pl.ds(j * 16, 16)]
                w = ew_v[pl.ds(j * 16, 16)]
                sig = 1.0 / (1.0 + jnp.exp(-w))
                plsc.addupdate_scatter(deg_v, [ii], sig)
                return c
            lax.fori_loop(0, DEG_CH // 16, acc_body, 0)
        pltpu.sync_copy(deg_v, out_hbm.at[pl.ds((t * NWORK + wid) * N, N)])


def _sc_agg_body(src_hbm, dst_hbm, ew_hbm, dinv_hbm, xw_hbm, out_hbm,
                 dinv_v, src_v, dst_v, nrm_v, rows_v, zer_v, acc_sh, sem):
    cid = lax.axis_index("c")
    sid = lax.axis_index("s")

    def zb(i, c):
        for cc in range(8):
            zer_v[i, pl.ds(cc * 16, 16)] = jnp.zeros((16,), jnp.float32)
        return c
    lax.fori_loop(0, ZR, zb, 0)

    tile0 = sid * RPTP
    for t in range(TT):
        for r5 in range(RPTP // ZR):
            pltpu.sync_copy(zer_v, acc_sh.at[pl.ds(tile0 + r5 * ZR, ZR)])
        pltpu.sync_copy(dinv_hbm.at[pl.ds(t * N, N)], dinv_v)
        plsc.subcore_barrier()
        base = t * E + cid * (E // 2) + sid * EPW

        def chunk(g, c):
            off = base + g * AGG_CH
            pltpu.sync_copy(src_hbm.at[pl.ds(off, AGG_CH)], src_v)
            pltpu.sync_copy(dst_hbm.at[pl.ds(off, AGG_CH)], dst_v)
            pltpu.sync_copy(ew_hbm.at[pl.ds(off, AGG_CH)], nrm_v)
            for j in range(AGG_CH // 16):
                sl = pl.ds(j * 16, 16)
                si = src_v[sl]
                w = nrm_v[sl]
                sig = 1.0 / (1.0 + jnp.exp(-w))
                dv = plsc.load_gather(dinv_v, [si])
                nrm_v[sl] = sig * dv
                src_v[sl] = si + t * N
            pltpu.async_copy(xw_hbm.at[src_v], rows_v, sem).wait()

            for gg in range(AGG_CH // 16):
                nj16 = nrm_v[pl.ds(gg * 16, 16)]
                for lane in range(16):
                    nj = nj16[lane]
                    row = gg * 16 + lane
                    for cc in range(8):
                        csl = pl.ds(cc * 16, 16)
                        rows_v[row, csl] = rows_v[row, csl] * nj
            pltpu.sync_copy(rows_v, acc_sh.at[dst_v], add=True)
            return c
        lax.fori_loop(0, EPW // AGG_CH, chunk, 0)
        plsc.subcore_barrier()
        for r5 in range(RPTP // ZR):
            sl = pl.ds(tile0 + r5 * ZR, ZR)
            pltpu.sync_copy(acc_sh.at[sl], out_hbm.at[t, cid, sl])
        plsc.subcore_barrier()


def kernel(X_list, edge_index_list, W_list, Y_hist_list, hidden_in,
           edge_weight_list, W_phi, b_phi, gc_W, gc_b, W_fuse, b_fuse,
           Wih, bih, Whh, bhh, Wd, bd, dw1, db1, dw2, db2):
    src = edge_index_list[:, 0, :].astype(jnp.int32).reshape(TT * E)
    dst = edge_index_list[:, 1, :].astype(jnp.int32).reshape(TT * E)
    ew = edge_weight_list.reshape(TT * E)

    sc_deg, sc_agg = _sc_kernels()
    phi, xw = _phi_xw(X_list, W_phi, b_phi.reshape(1, D), gc_W)
    deg_parts = sc_deg(dst, ew).reshape(TT, NWORK, N)
    dinv3, dd3 = _dinv(deg_parts)
    aggp = sc_agg(src, dst, ew, dinv3.reshape(TT * N),
                  xw.reshape(TT * N, D))
    dinv_col = dinv3.reshape(TT, N, 1)
    dd_col = dd3.reshape(TT, N, 1)

    wfp = W_fuse[:D]
    wfh = W_fuse[D:2 * D]
    wfr = W_fuse[2 * D:]
    wd_pad = jnp.zeros((D, D), jnp.float32).at[:, :GRID + 1].set(Wd)
    bd_pad = jnp.full((1, D), -1e30, jnp.float32).at[0, :GRID + 1].set(bd)
    dw1m = dw1.transpose(2, 0, 1)
    db1t = db1.T
    dw2f = dw2[:, 0, :]
    wihz = Wih[:, :D].T
    wihw = Wih[:, D][None]
    wihy = Wih[:, D + 1][None]
    whht = Whh.T

    h = hidden_in
    gcb3 = gc_b.reshape(TT, 1, D)
    ys, gs = [], []
    for t in range(TT):
        y_t, g_t, h = _stage2(
            t, phi, xw, aggp, dinv_col, dd_col, h, W_list, Y_hist_list,
            gcb3, wfp, wfh, wfr, b_fuse[None], wd_pad, bd_pad, dw1m, db1t,
            dw2f, db2, wihz, wihw, wihy, bih[None], whht, bhh[None])
        ys.append(y_t)
        gs.append(g_t[:, 0])
    return jnp.stack(ys), jnp.stack(gs), h


# speedup vs baseline: 20.9261x; 1.6864x over previous
import functools

import jax
import jax.numpy as jnp
from jax import lax
from jax.experimental import pallas as pl
from jax.experimental.pallas import tpu as pltpu
from jax.experimental.pallas import tpu_sc as plsc

N = 10000
E = 320000
TT = 4
D = 128
GRID = 10
NB = 1000
NBLK = N // NB
NWORK = 32
NP = 10240


def _phi_xw_body(x_ref, wphi_ref, bphi_ref, gcw_ref, phi_ref, xw_ref):
    x = x_ref[0]
    phi = jnp.maximum(
        jnp.dot(x, wphi_ref[...], preferred_element_type=jnp.float32)
        + bphi_ref[...], 0.0)
    phi_ref[0] = phi
    xw_ref[0] = jnp.dot(phi, gcw_ref[0], preferred_element_type=jnp.float32)


def _phi_xw(X, W_phi, b_phi2, gc_W):
    return pl.pallas_call(
        _phi_xw_body,
        grid=(TT, NBLK),
        in_specs=[
            pl.BlockSpec((1, NB, D), lambda t, b: (t, b, 0)),
            pl.BlockSpec((D, D), lambda t, b: (0, 0)),
            pl.BlockSpec((1, D), lambda t, b: (0, 0)),
            pl.BlockSpec((1, D, D), lambda t, b: (t, 0, 0)),
        ],
        out_specs=[
            pl.BlockSpec((1, NB, D), lambda t, b: (t, b, 0)),
            pl.BlockSpec((1, NB, D), lambda t, b: (t, b, 0)),
        ],
        out_shape=[
            jax.ShapeDtypeStruct((TT, N, D), jnp.float32),
            jax.ShapeDtypeStruct((TT, N, D), jnp.float32),
        ],
    )(X, W_phi, b_phi2, gc_W)


def _dinv_body(dp_ref, dinv_ref, dd_ref):
    deg = jnp.sum(dp_ref[0], axis=0, keepdims=True) + 1.0
    dinv = lax.rsqrt(deg)
    dinv_ref[0] = dinv
    dd_ref[0] = dinv * dinv


def _dinv(deg_parts):
    return pl.pallas_call(
        _dinv_body,
        grid=(TT,),
        in_specs=[pl.BlockSpec((1, NWORK, N), lambda t: (t, 0, 0))],
        out_specs=[
            pl.BlockSpec((1, 1, N), lambda t: (t, 0, 0)),
            pl.BlockSpec((1, 1, N), lambda t: (t, 0, 0)),
        ],
        out_shape=[
            jax.ShapeDtypeStruct((TT, 1, N), jnp.float32),
            jax.ShapeDtypeStruct((TT, 1, N), jnp.float32),
        ],
    )(deg_parts)


def _stage2_body(phi_ref, xw_ref, agg_ref, dinv_ref, dd_ref,
                 h_ref, w_ref, yh_ref, gcb_ref, wfp_ref, wfh_ref, wfr_ref,
                 bf_ref, wd_ref, bd_ref, dw1_ref, db1_ref, dw2_ref, db2_ref,
                 wihz_ref, wihw_ref, wihy_ref, bih_ref, whh_ref, bhh_ref,
                 y_ref, g_ref, hout_ref):
    f32 = jnp.float32
    phi = phi_ref[0]
    xwv = xw_ref[0]
    h = h_ref[...]
    agg = agg_ref[0, 0] + agg_ref[0, 1]
    rep = jnp.maximum(
        dinv_ref[0] * agg + dd_ref[0] * xwv + gcb_ref[0], 0.0)
    z = jnp.maximum(
        jnp.dot(phi, wfp_ref[...], preferred_element_type=f32)
        + jnp.dot(h, wfh_ref[...], preferred_element_type=f32)
        + jnp.dot(rep, wfr_ref[...], preferred_element_type=f32)
        + bf_ref[...], 0.0)

    logits = jnp.dot(z, wd_ref[...], preferred_element_type=f32) + bd_ref[...]
    m = jnp.max(logits, axis=1, keepdims=True)
    ex = jnp.exp(logits - m)
    sm = ex / jnp.sum(ex, axis=1, keepdims=True)
    treat = w_ref[0]
    tg = treat * float(GRID)
    U = jnp.ceil(tg)
    inter = 1.0 - (U - tg)
    L = U - 1.0
    L = L + (L < 0).astype(f32)
    Li = jnp.clip(L.astype(jnp.int32), 0, GRID)
    Ui = jnp.clip(U.astype(jnp.int32), 0, GRID)
    cols = lax.broadcasted_iota(jnp.int32, (NB, D), 1)
    Lout = jnp.sum(jnp.where(cols == Li, sm, 0.0), axis=1, keepdims=True)
    Uout = jnp.sum(jnp.where(cols == Ui, sm, 0.0), axis=1, keepdims=True)
    g_ref[...] = Lout + (Uout - Lout) * inter

    b0 = jnp.ones_like(treat)
    b1 = treat
    b2 = treat * treat
    b3 = jnp.maximum(treat - 0.33, 0.0) ** 2
    b4 = jnp.maximum(treat - 0.66, 0.0) ** 2
    basis5 = jnp.concatenate([b0, b1, b2, b3, b4], axis=1)
    o1 = (b0 * jnp.dot(z, dw1_ref[0], preferred_element_type=f32)
          + b1 * jnp.dot(z, dw1_ref[1], preferred_element_type=f32)
          + b2 * jnp.dot(z, dw1_ref[2], preferred_element_type=f32)
          + b3 * jnp.dot(z, dw1_ref[3], preferred_element_type=f32)
          + b4 * jnp.dot(z, dw1_ref[4], preferred_element_type=f32))
    o1 = o1 + jnp.dot(basis5, db1_ref[...], preferred_element_type=f32)
    o1 = jnp.maximum(o1, 0.0)
    y5 = jnp.dot(o1, dw2_ref[...], preferred_element_type=f32)
    y = (jnp.sum(basis5 * y5, axis=1, keepdims=True)
         + jnp.sum(basis5 * db2_ref[...], axis=1, keepdims=True))
    y_ref[...] = y

    gi = (jnp.dot(z, wihz_ref[...], preferred_element_type=f32)
          + treat * wihw_ref[...] + yh_ref[0] * wihy_ref[...]
          + bih_ref[...])
    gh = jnp.dot(h, whh_ref[...], preferred_element_type=f32) + bhh_ref[...]
    r = jax.nn.sigmoid(gi[:, :D] + gh[:, :D])
    zg = jax.nn.sigmoid(gi[:, D:2 * D] + gh[:, D:2 * D])
    n = jnp.tanh(gi[:, 2 * D:] + r * gh[:, 2 * D:])
    hout_ref[...] = (1.0 - zg) * n + zg * h


def _stage2(t, phi, xw, aggp, dinv_col, dd_col, h, W_list, Y_hist, gcb3,
            wfp, wfh, wfr, bf, wd, bd, dw1m, db1t, dw2f, db2r, wihz, wihw,
            wihy, bih2, whht, bhh2):
    tb = lambda b: (t, b, 0)
    nodeb = lambda b: (b, 0)
    full = lambda b: (0, 0)
    full3 = lambda b: (0, 0, 0)
    return pl.pallas_call(
        _stage2_body,
        grid=(NBLK,),
        in_specs=[
            pl.BlockSpec((1, NB, D), tb),
            pl.BlockSpec((1, NB, D), tb),
            pl.BlockSpec((1, 2, NB, D), lambda b: (t, 0, b, 0)),
            pl.BlockSpec((1, NB, 1), tb),
            pl.BlockSpec((1, NB, 1), tb),
            pl.BlockSpec((NB, D), nodeb),
            pl.BlockSpec((1, NB, 1), tb),
            pl.BlockSpec((1, NB, 1), tb),
            pl.BlockSpec((1, 1, D), lambda b: (t, 0, 0)),
            pl.BlockSpec((D, D), full),
            pl.BlockSpec((D, D), full),
            pl.BlockSpec((D, D), full),
            pl.BlockSpec((1, D), full),
            pl.BlockSpec((D, D), full),
            pl.BlockSpec((1, D), full),
            pl.BlockSpec((5, D, 64), full3),
            pl.BlockSpec((5, 64), full),
            pl.BlockSpec((64, 5), full),
            pl.BlockSpec((1, 5), full),
            pl.BlockSpec((D, 3 * D), full),
            pl.BlockSpec((1, 3 * D), full),
            pl.BlockSpec((1, 3 * D), full),
            pl.BlockSpec((1, 3 * D), full),
            pl.BlockSpec((D, 3 * D), full),
            pl.BlockSpec((1, 3 * D), full),
        ],
        out_specs=[
            pl.BlockSpec((NB, 1), nodeb),
            pl.BlockSpec((NB, 1), nodeb),
            pl.BlockSpec((NB, D), nodeb),
        ],
        out_shape=[
            jax.ShapeDtypeStruct((N, 1), jnp.float32),
            jax.ShapeDtypeStruct((N, 1), jnp.float32),
            jax.ShapeDtypeStruct((N, D), jnp.float32),
        ],
    )(phi, xw, aggp, dinv_col, dd_col, h, W_list, Y_hist, gcb3, wfp, wfh,
      wfr, bf, wd, bd, dw1m, db1t, dw2f, db2r, wihz, wihw, wihy, bih2,
      whht, bhh2)


EPW = E // NWORK
DEG_CH = 2000
AGG_CH = 80
RPTP = NP // 16
ZR = 80


@functools.cache
def _sc_kernels():
    mesh = plsc.VectorSubcoreMesh(core_axis_name="c", subcore_axis_name="s")
    params = pltpu.CompilerParams(needs_layout_passes=False)
    deg_k = functools.partial(
        pl.kernel,
        out_type=jax.ShapeDtypeStruct((TT * NWORK * N,), jnp.float32),
        mesh=mesh,
        compiler_params=params,
        scratch_types=[
            pltpu.VMEM((N,), jnp.float32),
            pltpu.VMEM((DEG_CH,), jnp.int32),
            pltpu.VMEM((DEG_CH,), jnp.float32),
        ],
    )(_sc_deg_body)
    agg_k = functools.partial(
        pl.kernel,
        out_type=jax.ShapeDtypeStruct((TT, 2, NP, D), jnp.float32),
        mesh=mesh,
        compiler_params=params,
        scratch_types=[
            pltpu.VMEM((N,), jnp.float32),
            pltpu.VMEM((AGG_CH,), jnp.int32),
            pltpu.VMEM((AGG_CH,), jnp.int32),
            pltpu.VMEM((AGG_CH,), jnp.int32),
            pltpu.VMEM((AGG_CH,), jnp.int32),
            pltpu.VMEM((AGG_CH,), jnp.float32),
            pltpu.VMEM((AGG_CH,), jnp.float32),
            pltpu.VMEM((AGG_CH, D), jnp.float32),
            pltpu.VMEM((AGG_CH, D), jnp.float32),
            pltpu.VMEM((ZR, D), jnp.float32),
            pltpu.VMEM_SHARED((NP, D), jnp.float32),
            pltpu.SemaphoreType.DMA,
            pltpu.SemaphoreType.DMA,
            pltpu.SemaphoreType.DMA,
            pltpu.SemaphoreType.DMA,
            pltpu.SemaphoreType.DMA,
            pltpu.SemaphoreType.DMA,
        ],
    )(_sc_agg_body)
    return deg_k, agg_k


def _sc_deg_body(dst_hbm, ew_hbm, out_hbm, deg_v, idx_v, ew_v):
    cid = lax.axis_index("c")
    sid = lax.axis_index("s")
    wid = sid * 2 + cid
    for t in range(TT):
        def zero_body(i, c):
            deg_v[pl.ds(i * 16, 16)] = jnp.zeros((16,), jnp.float32)
            return c
        lax.fori_loop(0, N // 16, zero_body, 0)
        base = t * E + wid * EPW
        for g in range(EPW // DEG_CH):
            sl = pl.ds(base + g * DEG_CH, DEG_CH)
            pltpu.sync_copy(dst_hbm.at[sl], idx_v)
            pltpu.sync_copy(ew_hbm.at[sl], ew_v)

            def acc_body(j, c):
                ii = idx_v[pl.ds(j * 16, 16)]
                w = ew_v[pl.ds(j * 16, 16)]
                sig = 1.0 / (1.0 + jnp.exp(-w))
                plsc.addupdate_scatter(deg_v, [ii], sig)
                return c
            lax.fori_loop(0, DEG_CH // 16, acc_body, 0)
        pltpu.sync_copy(deg_v, out_hbm.at[pl.ds((t * NWORK + wid) * N, N)])


def _sc_agg_body(src_hbm, dst_hbm, ew_hbm, dinv_hbm, xw_hbm, out_hbm,
                 dinv_v, src0, src1, dst0, dst1, nrm0, nrm1, rows0, rows1,
                 zer_v, acc_sh, sidx0, sidx1, sgat0, sgat1, ssc0, ssc1):
    cid = lax.axis_index("c")
    sid = lax.axis_index("s")
    NCH = EPW // AGG_CH
    bufs = ((src0, dst0, nrm0, rows0, sidx0, sgat0, ssc0),
            (src1, dst1, nrm1, rows1, sidx1, sgat1, ssc1))

    def zb(i, c):
        for cc in range(8):
            zer_v[i, pl.ds(cc * 16, 16)] = jnp.zeros((16,), jnp.float32)
        return c
    lax.fori_loop(0, ZR, zb, 0)

    tile0 = sid * RPTP

    def tbody(t, tc):
        base = t * E + cid * (E // 2) + sid * EPW

        def fire_idx(g, b):
            src_v, dst_v, nrm_v, _, sidx, _, _ = bufs[b]
            off = base + g * AGG_CH
            pltpu.async_copy(src_hbm.at[pl.ds(off, AGG_CH)], src_v, sidx)
            pltpu.async_copy(dst_hbm.at[pl.ds(off, AGG_CH)], dst_v, sidx)
            pltpu.async_copy(ew_hbm.at[pl.ds(off, AGG_CH)], nrm_v, sidx)

        def wait_idx(g, b):
            src_v, dst_v, nrm_v, _, sidx, _, _ = bufs[b]
            off = base + g * AGG_CH
            sl = pl.ds(off, AGG_CH)
            pltpu.make_async_copy(src_hbm.at[sl], src_v, sidx).wait()
            pltpu.make_async_copy(dst_hbm.at[sl], dst_v, sidx).wait()
            pltpu.make_async_copy(ew_hbm.at[sl], nrm_v, sidx).wait()

        def norm_and_gather(b):
            src_v, _, nrm_v, rows_v, _, sgat, _ = bufs[b]
            for j in range(AGG_CH // 16):
                sl = pl.ds(j * 16, 16)
                si = src_v[sl]
                w = nrm_v[sl]
                sig = 1.0 / (1.0 + jnp.exp(-w))
                dv = plsc.load_gather(dinv_v, [si])
                nrm_v[sl] = sig * dv
            pltpu.async_copy(xw_hbm.at[t].at[src_v], rows_v, sgat)

        def scale_and_scatter(b):
            src_v, dst_v, nrm_v, rows_v, _, sgat, ssc = bufs[b]
            pltpu.make_async_copy(xw_hbm.at[t].at[src_v], rows_v, sgat).wait()
            for gg in range(AGG_CH // 16):
                nj16 = nrm_v[pl.ds(gg * 16, 16)]
                for lane in range(16):
                    nj = nj16[lane]
                    row = gg * 16 + lane
                    for cc in range(8):
                        csl = pl.ds(cc * 16, 16)
                        rows_v[row, csl] = rows_v[row, csl] * nj
            pltpu.async_copy(rows_v, acc_sh.at[dst_v], ssc, add=True)

        def wait_scatter(b):
            _, dst_v, _, rows_v, _, _, ssc = bufs[b]
            pltpu.make_async_copy(rows_v, acc_sh.at[dst_v], ssc).wait()

        for r5 in range(RPTP // ZR):
            pltpu.sync_copy(zer_v, acc_sh.at[pl.ds(tile0 + r5 * ZR, ZR)])
        pltpu.sync_copy(dinv_hbm.at[pl.ds(t * N, N)], dinv_v)
        plsc.subcore_barrier()

        fire_idx(0, 0)
        wait_idx(0, 0)
        norm_and_gather(0)

        nhalf = (NCH - 1) // 2

        def body(i, c):
            @pl.when(i > 0)
            def _():
                wait_scatter(1)

            @pl.when(i < nhalf)
            def _():
                fire_idx(2 * i + 1, 1)
            scale_and_scatter(0)

            @pl.when(i < nhalf)
            def _():
                wait_idx(2 * i + 1, 1)
                norm_and_gather(1)
                wait_scatter(0)
                fire_idx(2 * i + 2, 0)
                scale_and_scatter(1)
                wait_idx(2 * i + 2, 0)
                norm_and_gather(0)
            return c
        lax.fori_loop(0, nhalf + 1, body, 0)

        wait_scatter(0)
        plsc.subcore_barrier()
        for r5 in range(RPTP // ZR):
            sl = pl.ds(tile0 + r5 * ZR, ZR)
            pltpu.sync_copy(acc_sh.at[sl], out_hbm.at[t, cid, sl])
        plsc.subcore_barrier()
        return tc

    lax.fori_loop(0, TT, tbody, 0)


def kernel(X_list, edge_index_list, W_list, Y_hist_list, hidden_in,
           edge_weight_list, W_phi, b_phi, gc_W, gc_b, W_fuse, b_fuse,
           Wih, bih, Whh, bhh, Wd, bd, dw1, db1, dw2, db2):
    src = edge_index_list[:, 0, :].astype(jnp.int32).reshape(TT * E)
    dst = edge_index_list[:, 1, :].astype(jnp.int32).reshape(TT * E)
    ew = edge_weight_list.reshape(TT * E)

    sc_deg, sc_agg = _sc_kernels()
    phi, xw = _phi_xw(X_list, W_phi, b_phi.reshape(1, D), gc_W)
    deg_parts = sc_deg(dst, ew).reshape(TT, NWORK, N)
    dinv3, dd3 = _dinv(deg_parts)
    aggp = sc_agg(src, dst, ew, dinv3.reshape(TT * N), xw)
    dinv_col = dinv3.reshape(TT, N, 1)
    dd_col = dd3.reshape(TT, N, 1)

    wfp = W_fuse[:D]
    wfh = W_fuse[D:2 * D]
    wfr = W_fuse[2 * D:]
    wd_pad = jnp.zeros((D, D), jnp.float32).at[:, :GRID + 1].set(Wd)
    bd_pad = jnp.full((1, D), -1e30, jnp.float32).at[0, :GRID + 1].set(bd)
    dw1m = dw1.transpose(2, 0, 1)
    db1t = db1.T
    dw2f = dw2[:, 0, :]
    wihz = Wih[:, :D].T
    wihw = Wih[:, D][None]
    wihy = Wih[:, D + 1][None]
    whht = Whh.T

    h = hidden_in
    gcb3 = gc_b.reshape(TT, 1, D)
    ys, gs = [], []
    for t in range(TT):
        y_t, g_t, h = _stage2(
            t, phi, xw, aggp, dinv_col, dd_col, h, W_list, Y_hist_list,
            gcb3, wfp, wfh, wfr, b_fuse[None], wd_pad, bd_pad, dw1m, db1t,
            dw2f, db2, wihz, wihw, wihy, bih[None], whht, bhh[None])
        ys.append(y_t)
        gs.append(g_t[:, 0])
    return jnp.stack(ys), jnp.stack(gs), h


# speedup vs baseline: 28.5763x; 1.3656x over previous
import functools

import jax
import jax.numpy as jnp
from jax import lax
from jax.experimental import pallas as pl
from jax.experimental.pallas import tpu as pltpu
from jax.experimental.pallas import tpu_sc as plsc

N = 10000
E = 320000
TT = 4
D = 128
GRID = 10
NB = 1000
NBLK = N // NB
NWORK = 32
NP = 10240


def _phi_xw_body(x_ref, wphi_ref, bphi_ref, gcw_ref, phi_ref, xw_ref):
    x = x_ref[0]
    phi = jnp.maximum(
        jnp.dot(x, wphi_ref[...], preferred_element_type=jnp.float32)
        + bphi_ref[...], 0.0)
    phi_ref[0] = phi
    xw_ref[0] = jnp.dot(phi, gcw_ref[0], preferred_element_type=jnp.float32)


def _phi_xw(X, W_phi, b_phi2, gc_W):
    return pl.pallas_call(
        _phi_xw_body,
        grid=(TT, NBLK),
        in_specs=[
            pl.BlockSpec((1, NB, D), lambda t, b: (t, b, 0)),
            pl.BlockSpec((D, D), lambda t, b: (0, 0)),
            pl.BlockSpec((1, D), lambda t, b: (0, 0)),
            pl.BlockSpec((1, D, D), lambda t, b: (t, 0, 0)),
        ],
        out_specs=[
            pl.BlockSpec((1, NB, D), lambda t, b: (t, b, 0)),
            pl.BlockSpec((1, NB, D), lambda t, b: (t, b, 0)),
        ],
        out_shape=[
            jax.ShapeDtypeStruct((TT, N, D), jnp.float32),
            jax.ShapeDtypeStruct((TT, N, D), jnp.float32),
        ],
    )(X, W_phi, b_phi2, gc_W)


def _dinv_body(dp_ref, dinv_ref, dd_ref):
    deg = jnp.sum(dp_ref[0], axis=0, keepdims=True) + 1.0
    dinv = lax.rsqrt(deg)
    dinv_ref[0] = dinv
    dd_ref[0] = dinv * dinv


def _dinv(deg_parts):
    return pl.pallas_call(
        _dinv_body,
        grid=(TT,),
        in_specs=[pl.BlockSpec((1, NWORK, N), lambda t: (t, 0, 0))],
        out_specs=[
            pl.BlockSpec((1, 1, N), lambda t: (t, 0, 0)),
            pl.BlockSpec((1, 1, N), lambda t: (t, 0, 0)),
        ],
        out_shape=[
            jax.ShapeDtypeStruct((TT, 1, N), jnp.float32),
            jax.ShapeDtypeStruct((TT, 1, N), jnp.float32),
        ],
    )(deg_parts)


def _stage2_body(phi_ref, xw_ref, agg_ref, dinv_ref, dd_ref,
                 h_ref, w_ref, yh_ref, gcb_ref, wfp_ref, wfh_ref, wfr_ref,
                 bf_ref, wd_ref, bd_ref, dw1_ref, db1_ref, dw2_ref, db2_ref,
                 wihz_ref, wihw_ref, wihy_ref, bih_ref, whh_ref, bhh_ref,
                 y_ref, g_ref, hout_ref):
    f32 = jnp.float32
    phi = phi_ref[0]
    xwv = xw_ref[0]
    h = h_ref[...]
    agg = agg_ref[0, 0] + agg_ref[0, 1]
    rep = jnp.maximum(
        dinv_ref[0] * agg + dd_ref[0] * xwv + gcb_ref[0], 0.0)
    z = jnp.maximum(
        jnp.dot(phi, wfp_ref[...], preferred_element_type=f32)
        + jnp.dot(h, wfh_ref[...], preferred_element_type=f32)
        + jnp.dot(rep, wfr_ref[...], preferred_element_type=f32)
        + bf_ref[...], 0.0)

    logits = jnp.dot(z, wd_ref[...], preferred_element_type=f32) + bd_ref[...]
    m = jnp.max(logits, axis=1, keepdims=True)
    ex = jnp.exp(logits - m)
    sm = ex / jnp.sum(ex, axis=1, keepdims=True)
    treat = w_ref[0]
    tg = treat * float(GRID)
    U = jnp.ceil(tg)
    inter = 1.0 - (U - tg)
    L = U - 1.0
    L = L + (L < 0).astype(f32)
    Li = jnp.clip(L.astype(jnp.int32), 0, GRID)
    Ui = jnp.clip(U.astype(jnp.int32), 0, GRID)
    cols = lax.broadcasted_iota(jnp.int32, (NB, D), 1)
    Lout = jnp.sum(jnp.where(cols == Li, sm, 0.0), axis=1, keepdims=True)
    Uout = jnp.sum(jnp.where(cols == Ui, sm, 0.0), axis=1, keepdims=True)
    g_ref[...] = Lout + (Uout - Lout) * inter

    b0 = jnp.ones_like(treat)
    b1 = treat
    b2 = treat * treat
    b3 = jnp.maximum(treat - 0.33, 0.0) ** 2
    b4 = jnp.maximum(treat - 0.66, 0.0) ** 2
    basis5 = jnp.concatenate([b0, b1, b2, b3, b4], axis=1)
    o1 = (b0 * jnp.dot(z, dw1_ref[0], preferred_element_type=f32)
          + b1 * jnp.dot(z, dw1_ref[1], preferred_element_type=f32)
          + b2 * jnp.dot(z, dw1_ref[2], preferred_element_type=f32)
          + b3 * jnp.dot(z, dw1_ref[3], preferred_element_type=f32)
          + b4 * jnp.dot(z, dw1_ref[4], preferred_element_type=f32))
    o1 = o1 + jnp.dot(basis5, db1_ref[...], preferred_element_type=f32)
    o1 = jnp.maximum(o1, 0.0)
    y5 = jnp.dot(o1, dw2_ref[...], preferred_element_type=f32)
    y = (jnp.sum(basis5 * y5, axis=1, keepdims=True)
         + jnp.sum(basis5 * db2_ref[...], axis=1, keepdims=True))
    y_ref[...] = y

    gi = (jnp.dot(z, wihz_ref[...], preferred_element_type=f32)
          + treat * wihw_ref[...] + yh_ref[0] * wihy_ref[...]
          + bih_ref[...])
    gh = jnp.dot(h, whh_ref[...], preferred_element_type=f32) + bhh_ref[...]
    r = jax.nn.sigmoid(gi[:, :D] + gh[:, :D])
    zg = jax.nn.sigmoid(gi[:, D:2 * D] + gh[:, D:2 * D])
    n = jnp.tanh(gi[:, 2 * D:] + r * gh[:, 2 * D:])
    hout_ref[...] = (1.0 - zg) * n + zg * h


def _stage2(t, phi, xw, aggp, dinv_col, dd_col, h, W_list, Y_hist, gcb3,
            wfp, wfh, wfr, bf, wd, bd, dw1m, db1t, dw2f, db2r, wihz, wihw,
            wihy, bih2, whht, bhh2):
    tb = lambda b: (t, b, 0)
    nodeb = lambda b: (b, 0)
    full = lambda b: (0, 0)
    full3 = lambda b: (0, 0, 0)
    return pl.pallas_call(
        _stage2_body,
        grid=(NBLK,),
        in_specs=[
            pl.BlockSpec((1, NB, D), tb),
            pl.BlockSpec((1, NB, D), tb),
            pl.BlockSpec((1, 2, NB, D), lambda b: (t, 0, b, 0)),
            pl.BlockSpec((1, NB, 1), tb),
            pl.BlockSpec((1, NB, 1), tb),
            pl.BlockSpec((NB, D), nodeb),
            pl.BlockSpec((1, NB, 1), tb),
            pl.BlockSpec((1, NB, 1), tb),
            pl.BlockSpec((1, 1, D), lambda b: (t, 0, 0)),
            pl.BlockSpec((D, D), full),
            pl.BlockSpec((D, D), full),
            pl.BlockSpec((D, D), full),
            pl.BlockSpec((1, D), full),
            pl.BlockSpec((D, D), full),
            pl.BlockSpec((1, D), full),
            pl.BlockSpec((5, D, 64), full3),
            pl.BlockSpec((5, 64), full),
            pl.BlockSpec((64, 5), full),
            pl.BlockSpec((1, 5), full),
            pl.BlockSpec((D, 3 * D), full),
            pl.BlockSpec((1, 3 * D), full),
            pl.BlockSpec((1, 3 * D), full),
            pl.BlockSpec((1, 3 * D), full),
            pl.BlockSpec((D, 3 * D), full),
            pl.BlockSpec((1, 3 * D), full),
        ],
        out_specs=[
            pl.BlockSpec((NB, 1), nodeb),
            pl.BlockSpec((NB, 1), nodeb),
            pl.BlockSpec((NB, D), nodeb),
        ],
        out_shape=[
            jax.ShapeDtypeStruct((N, 1), jnp.float32),
            jax.ShapeDtypeStruct((N, 1), jnp.float32),
            jax.ShapeDtypeStruct((N, D), jnp.float32),
        ],
    )(phi, xw, aggp, dinv_col, dd_col, h, W_list, Y_hist, gcb3, wfp, wfh,
      wfr, bf, wd, bd, dw1m, db1t, dw2f, db2r, wihz, wihw, wihy, bih2,
      whht, bhh2)


EPW = E // NWORK
DEG_CH = 2000
AGG_CH = 80
RPTP = NP // 16
ZR = 80


@functools.cache
def _sc_kernels():
    mesh = plsc.VectorSubcoreMesh(core_axis_name="c", subcore_axis_name="s")
    params = pltpu.CompilerParams(needs_layout_passes=False)
    deg_k = functools.partial(
        pl.kernel,
        out_type=jax.ShapeDtypeStruct((TT * NWORK * N,), jnp.float32),
        mesh=mesh,
        compiler_params=params,
        scratch_types=[
            pltpu.VMEM((N,), jnp.float32),
            pltpu.VMEM((DEG_CH,), jnp.int32),
            pltpu.VMEM((DEG_CH,), jnp.float32),
        ],
    )(_sc_deg_body)
    agg_k = functools.partial(
        pl.kernel,
        out_type=jax.ShapeDtypeStruct((TT, 2, NP, D), jnp.float32),
        mesh=mesh,
        compiler_params=params,
        scratch_types=(
            [pltpu.VMEM((N,), jnp.float32)]
            + [pltpu.VMEM((AGG_CH,), jnp.int32) for _ in range(3)]
            + [pltpu.VMEM((AGG_CH,), jnp.int32) for _ in range(3)]
            + [pltpu.VMEM((AGG_CH,), jnp.float32) for _ in range(3)]
            + [pltpu.VMEM((AGG_CH,), jnp.int32) for _ in range(3)]
            + [pltpu.VMEM((AGG_CH,), jnp.int32) for _ in range(3)]
            + [pltpu.VMEM((AGG_CH,), jnp.float32) for _ in range(3)]
            + [pltpu.VMEM((AGG_CH, D), jnp.float32) for _ in range(3)]
            + [pltpu.VMEM_SHARED((NP, D), jnp.float32)]
            + [pltpu.SemaphoreType.DMA for _ in range(9)]
        ),
    )(_sc_agg_body)
    return deg_k, agg_k


def _sc_deg_body(dst_hbm, ew_hbm, out_hbm, deg_v, idx_v, ew_v):
    cid = lax.axis_index("c")
    sid = lax.axis_index("s")
    wid = sid * 2 + cid
    for t in range(TT):
        def zero_body(i, c):
            deg_v[pl.ds(i * 16, 16)] = jnp.zeros((16,), jnp.float32)
            return c
        lax.fori_loop(0, N // 16, zero_body, 0)
        base = t * E + wid * EPW
        for g in range(EPW // DEG_CH):
            sl = pl.ds(base + g * DEG_CH, DEG_CH)
            pltpu.sync_copy(dst_hbm.at[sl], idx_v)
            pltpu.sync_copy(ew_hbm.at[sl], ew_v)

            def acc_body(j, c):
                ii = idx_v[pl.ds(j * 16, 16)]
                w = ew_v[pl.ds(j * 16, 16)]
                sig = 1.0 / (1.0 + jnp.exp(-w))
                plsc.addupdate_scatter(deg_v, [ii], sig)
                return c
            lax.fori_loop(0, DEG_CH // 16, acc_body, 0)
        pltpu.sync_copy(deg_v, out_hbm.at[pl.ds((t * NWORK + wid) * N, N)])


def _sc_agg_body(src_hbm, dst_hbm, ew_hbm, dinv_hbm, xw_hbm, zer_hbm,
                 out_hbm, dinv_v, *sc):
    cid = lax.axis_index("c")
    sid = lax.axis_index("s")
    NCH = EPW // AGG_CH
    sraw, draw, eraw = sc[0:3], sc[3:6], sc[6:9]
    gsrc, gdst, nrm = sc[9:12], sc[12:15], sc[15:18]
    rows = sc[18:21]
    acc_sh = sc[21]
    sidx, sgat, ssc = sc[22:25], sc[25:28], sc[28:31]

    tile0 = sid * RPTP

    def tbody(t, tc):
        base = t * E + cid * (E // 2) + sid * EPW

        def fire_idx(g, b):
            off = base + g * AGG_CH
            sl = pl.ds(off, AGG_CH)
            pltpu.async_copy(src_hbm.at[sl], sraw[b], sidx[b])
            pltpu.async_copy(dst_hbm.at[sl], draw[b], sidx[b])
            pltpu.async_copy(ew_hbm.at[sl], eraw[b], sidx[b])

        def wait_idx(g, b):
            off = base + g * AGG_CH
            sl = pl.ds(off, AGG_CH)
            pltpu.make_async_copy(src_hbm.at[sl], sraw[b], sidx[b]).wait()
            pltpu.make_async_copy(dst_hbm.at[sl], draw[b], sidx[b]).wait()
            pltpu.make_async_copy(ew_hbm.at[sl], eraw[b], sidx[b]).wait()

        def prep(b):
            for j in range(AGG_CH // 16):
                sl = pl.ds(j * 16, 16)
                si = sraw[b][sl]
                w = eraw[b][sl]
                sig = 1.0 / (1.0 + jnp.exp(-w))
                dv = plsc.load_gather(dinv_v, [si])
                nrm[b][sl] = sig * dv
                gsrc[b][sl] = si
                gdst[b][sl] = draw[b][sl]
            pltpu.async_copy(xw_hbm.at[t].at[gsrc[b]], rows[b], sgat[b])

        def wait_gather(b):
            pltpu.make_async_copy(
                xw_hbm.at[t].at[gsrc[b]], rows[b], sgat[b]).wait()

        def scale(b):
            def sg(jg, c):
                nj16 = nrm[b][pl.ds(jg * 16, 16)]
                for lane in range(16):
                    nj = nj16[lane]
                    row = jg * 16 + lane
                    for cc in range(8):
                        csl = pl.ds(cc * 16, 16)
                        rows[b][row, csl] = rows[b][row, csl] * nj
                return c
            lax.fori_loop(0, AGG_CH // 16, sg, 0)

        def fire_scatter(b):
            pltpu.async_copy(rows[b], acc_sh.at[gdst[b]], ssc[b], add=True)

        def wait_scatter(b):
            pltpu.make_async_copy(rows[b], acc_sh.at[gdst[b]], ssc[b]).wait()

        pltpu.sync_copy(zer_hbm, acc_sh.at[pl.ds(tile0, RPTP)])
        pltpu.sync_copy(dinv_hbm.at[pl.ds(t * N, N)], dinv_v)
        plsc.subcore_barrier()

        fire_idx(0, 0)
        wait_idx(0, 0)
        prep(0)
        fire_idx(1, 1)

        def body(i, c):
            for k in range(3):
                g = 3 * i + k
                b, bn, bnn = k, (k + 1) % 3, (k + 2) % 3
                if k < 2:
                    @pl.when(i > 0)
                    def _():
                        wait_scatter(bn)
                else:
                    wait_scatter(bn)
                wait_idx(g + 1, bn)
                prep(bn)
                fire_idx(g + 2, bnn)
                wait_gather(b)
                scale(b)
                fire_scatter(b)
            return c
        lax.fori_loop(0, (NCH - 2) // 3, body, 0)

        wait_scatter(1)
        wait_idx(NCH - 1, 1)
        prep(1)
        wait_gather(0)
        scale(0)
        fire_scatter(0)
        wait_scatter(2)
        wait_gather(1)
        scale(1)
        fire_scatter(1)
        wait_scatter(0)
        wait_scatter(1)
        plsc.subcore_barrier()
        sl = pl.ds(tile0, RPTP)
        pltpu.sync_copy(acc_sh.at[sl], out_hbm.at[t, cid, sl])
        plsc.subcore_barrier()
        return tc

    lax.fori_loop(0, TT, tbody, 0)


def kernel(X_list, edge_index_list, W_list, Y_hist_list, hidden_in,
           edge_weight_list, W_phi, b_phi, gc_W, gc_b, W_fuse, b_fuse,
           Wih, bih, Whh, bhh, Wd, bd, dw1, db1, dw2, db2):
    src = edge_index_list[:, 0, :].astype(jnp.int32).reshape(TT * E)
    dst = edge_index_list[:, 1, :].astype(jnp.int32).reshape(TT * E)
    ew = edge_weight_list.reshape(TT * E)

    sc_deg, sc_agg = _sc_kernels()
    phi, xw = _phi_xw(X_list, W_phi, b_phi.reshape(1, D), gc_W)
    deg_parts = sc_deg(dst, ew).reshape(TT, NWORK, N)
    dinv3, dd3 = _dinv(deg_parts)
    aggp = sc_agg(src, dst, ew, dinv3.reshape(TT * N), xw,
                  jnp.zeros((RPTP, D), jnp.float32))
    dinv_col = dinv3.reshape(TT, N, 1)
    dd_col = dd3.reshape(TT, N, 1)

    wfp = W_fuse[:D]
    wfh = W_fuse[D:2 * D]
    wfr = W_fuse[2 * D:]
    wd_pad = jnp.zeros((D, D), jnp.float32).at[:, :GRID + 1].set(Wd)
    bd_pad = jnp.full((1, D), -1e30, jnp.float32).at[0, :GRID + 1].set(bd)
    dw1m = dw1.transpose(2, 0, 1)
    db1t = db1.T
    dw2f = dw2[:, 0, :]
    wihz = Wih[:, :D].T
    wihw = Wih[:, D][None]
    wihy = Wih[:, D + 1][None]
    whht = Whh.T

    h = hidden_in
    gcb3 = gc_b.reshape(TT, 1, D)
    ys, gs = [], []
    for t in range(TT):
        y_t, g_t, h = _stage2(
            t, phi, xw, aggp, dinv_col, dd_col, h, W_list, Y_hist_list,
            gcb3, wfp, wfh, wfr, b_fuse[None], wd_pad, bd_pad, dw1m, db1t,
            dw2f, db2, wihz, wihw, wihy, bih[None], whht, bhh[None])
        ys.append(y_t)
        gs.append(g_t[:, 0])
    return jnp.stack(ys), jnp.stack(gs), h


# speedup vs baseline: 28.9761x; 1.0140x over previous
import functools

import jax
import jax.numpy as jnp
from jax import lax
from jax.experimental import pallas as pl
from jax.experimental.pallas import tpu as pltpu
from jax.experimental.pallas import tpu_sc as plsc

N = 10000
E = 320000
TT = 4
D = 128
GRID = 10
NB = 1000
NBLK = N // NB
NWORK = 32
NP = 10240


def _phi_xw_body(x_ref, wphi_ref, bphi_ref, gcw_ref, phi_ref, xw_ref):
    x = x_ref[0]
    phi = jnp.maximum(
        jnp.dot(x, wphi_ref[...], preferred_element_type=jnp.float32)
        + bphi_ref[...], 0.0)
    phi_ref[0] = phi
    xw_ref[0] = jnp.dot(phi, gcw_ref[0], preferred_element_type=jnp.float32)


def _phi_xw(X, W_phi, b_phi2, gc_W):
    return pl.pallas_call(
        _phi_xw_body,
        grid=(TT, NBLK),
        in_specs=[
            pl.BlockSpec((1, NB, D), lambda t, b: (t, b, 0)),
            pl.BlockSpec((D, D), lambda t, b: (0, 0)),
            pl.BlockSpec((1, D), lambda t, b: (0, 0)),
            pl.BlockSpec((1, D, D), lambda t, b: (t, 0, 0)),
        ],
        out_specs=[
            pl.BlockSpec((1, NB, D), lambda t, b: (t, b, 0)),
            pl.BlockSpec((1, NB, D), lambda t, b: (t, b, 0)),
        ],
        out_shape=[
            jax.ShapeDtypeStruct((TT, N, D), jnp.float32),
            jax.ShapeDtypeStruct((TT, N, D), jnp.float32),
        ],
    )(X, W_phi, b_phi2, gc_W)


def _dinv_body(dp_ref, dinv_ref, dd_ref):
    deg = jnp.sum(dp_ref[0], axis=0, keepdims=True) + 1.0
    dinv = lax.rsqrt(deg)
    dinv_ref[0] = dinv
    dd_ref[0] = dinv * dinv


def _dinv(deg_parts):
    return pl.pallas_call(
        _dinv_body,
        grid=(TT,),
        in_specs=[pl.BlockSpec((1, NWORK, N), lambda t: (t, 0, 0))],
        out_specs=[
            pl.BlockSpec((1, 1, N), lambda t: (t, 0, 0)),
            pl.BlockSpec((1, 1, N), lambda t: (t, 0, 0)),
        ],
        out_shape=[
            jax.ShapeDtypeStruct((TT, 1, N), jnp.float32),
            jax.ShapeDtypeStruct((TT, 1, N), jnp.float32),
        ],
    )(deg_parts)


def _stage2_body(phi_ref, xw_ref, agg_ref, dinv_ref, dd_ref,
                 h_ref, w_ref, yh_ref, gcb_ref, wfp_ref, wfh_ref, wfr_ref,
                 bf_ref, wd_ref, bd_ref, dw1_ref, db1_ref, dw2_ref, db2_ref,
                 wihz_ref, wihw_ref, wihy_ref, bih_ref, whh_ref, bhh_ref,
                 y_ref, g_ref, hout_ref):
    f32 = jnp.float32
    phi = phi_ref[0]
    xwv = xw_ref[0]
    h = h_ref[...]
    agg = agg_ref[0, 0] + agg_ref[0, 1]
    rep = jnp.maximum(
        dinv_ref[0] * agg + dd_ref[0] * xwv + gcb_ref[0], 0.0)
    z = jnp.maximum(
        jnp.dot(phi, wfp_ref[...], preferred_element_type=f32)
        + jnp.dot(h, wfh_ref[...], preferred_element_type=f32)
        + jnp.dot(rep, wfr_ref[...], preferred_element_type=f32)
        + bf_ref[...], 0.0)

    logits = jnp.dot(z, wd_ref[...], preferred_element_type=f32) + bd_ref[...]
    m = jnp.max(logits, axis=1, keepdims=True)
    ex = jnp.exp(logits - m)
    sm = ex / jnp.sum(ex, axis=1, keepdims=True)
    treat = w_ref[0]
    tg = treat * float(GRID)
    U = jnp.ceil(tg)
    inter = 1.0 - (U - tg)
    L = U - 1.0
    L = L + (L < 0).astype(f32)
    Li = jnp.clip(L.astype(jnp.int32), 0, GRID)
    Ui = jnp.clip(U.astype(jnp.int32), 0, GRID)
    cols = lax.broadcasted_iota(jnp.int32, (NB, D), 1)
    Lout = jnp.sum(jnp.where(cols == Li, sm, 0.0), axis=1, keepdims=True)
    Uout = jnp.sum(jnp.where(cols == Ui, sm, 0.0), axis=1, keepdims=True)
    g_ref[...] = Lout + (Uout - Lout) * inter

    b0 = jnp.ones_like(treat)
    b1 = treat
    b2 = treat * treat
    b3 = jnp.maximum(treat - 0.33, 0.0) ** 2
    b4 = jnp.maximum(treat - 0.66, 0.0) ** 2
    basis5 = jnp.concatenate([b0, b1, b2, b3, b4], axis=1)
    o1 = (b0 * jnp.dot(z, dw1_ref[0], preferred_element_type=f32)
          + b1 * jnp.dot(z, dw1_ref[1], preferred_element_type=f32)
          + b2 * jnp.dot(z, dw1_ref[2], preferred_element_type=f32)
          + b3 * jnp.dot(z, dw1_ref[3], preferred_element_type=f32)
          + b4 * jnp.dot(z, dw1_ref[4], preferred_element_type=f32))
    o1 = o1 + jnp.dot(basis5, db1_ref[...], preferred_element_type=f32)
    o1 = jnp.maximum(o1, 0.0)
    y5 = jnp.dot(o1, dw2_ref[...], preferred_element_type=f32)
    y = (jnp.sum(basis5 * y5, axis=1, keepdims=True)
         + jnp.sum(basis5 * db2_ref[...], axis=1, keepdims=True))
    y_ref[...] = y

    gi = (jnp.dot(z, wihz_ref[...], preferred_element_type=f32)
          + treat * wihw_ref[...] + yh_ref[0] * wihy_ref[...]
          + bih_ref[...])
    gh = jnp.dot(h, whh_ref[...], preferred_element_type=f32) + bhh_ref[...]
    r = jax.nn.sigmoid(gi[:, :D] + gh[:, :D])
    zg = jax.nn.sigmoid(gi[:, D:2 * D] + gh[:, D:2 * D])
    n = jnp.tanh(gi[:, 2 * D:] + r * gh[:, 2 * D:])
    hout_ref[...] = (1.0 - zg) * n + zg * h


def _stage2(t, phi, xw, aggp, dinv_col, dd_col, h, W_list, Y_hist, gcb3,
            wfp, wfh, wfr, bf, wd, bd, dw1m, db1t, dw2f, db2r, wihz, wihw,
            wihy, bih2, whht, bhh2):
    tb = lambda b: (t, b, 0)
    nodeb = lambda b: (b, 0)
    full = lambda b: (0, 0)
    full3 = lambda b: (0, 0, 0)
    return pl.pallas_call(
        _stage2_body,
        grid=(NBLK,),
        in_specs=[
            pl.BlockSpec((1, NB, D), tb),
            pl.BlockSpec((1, NB, D), tb),
            pl.BlockSpec((1, 2, NB, D), lambda b: (t, 0, b, 0)),
            pl.BlockSpec((1, NB, 1), tb),
            pl.BlockSpec((1, NB, 1), tb),
            pl.BlockSpec((NB, D), nodeb),
            pl.BlockSpec((1, NB, 1), tb),
            pl.BlockSpec((1, NB, 1), tb),
            pl.BlockSpec((1, 1, D), lambda b: (t, 0, 0)),
            pl.BlockSpec((D, D), full),
            pl.BlockSpec((D, D), full),
            pl.BlockSpec((D, D), full),
            pl.BlockSpec((1, D), full),
            pl.BlockSpec((D, D), full),
            pl.BlockSpec((1, D), full),
            pl.BlockSpec((5, D, 64), full3),
            pl.BlockSpec((5, 64), full),
            pl.BlockSpec((64, 5), full),
            pl.BlockSpec((1, 5), full),
            pl.BlockSpec((D, 3 * D), full),
            pl.BlockSpec((1, 3 * D), full),
            pl.BlockSpec((1, 3 * D), full),
            pl.BlockSpec((1, 3 * D), full),
            pl.BlockSpec((D, 3 * D), full),
            pl.BlockSpec((1, 3 * D), full),
        ],
        out_specs=[
            pl.BlockSpec((NB, 1), nodeb),
            pl.BlockSpec((NB, 1), nodeb),
            pl.BlockSpec((NB, D), nodeb),
        ],
        out_shape=[
            jax.ShapeDtypeStruct((N, 1), jnp.float32),
            jax.ShapeDtypeStruct((N, 1), jnp.float32),
            jax.ShapeDtypeStruct((N, D), jnp.float32),
        ],
    )(phi, xw, aggp, dinv_col, dd_col, h, W_list, Y_hist, gcb3, wfp, wfh,
      wfr, bf, wd, bd, dw1m, db1t, dw2f, db2r, wihz, wihw, wihy, bih2,
      whht, bhh2)


EPW = E // NWORK
DEG_CH = 2000
AGG_CH = 80
RPTP = NP // 16
ZR = 80


@functools.cache
def _sc_kernels():
    mesh = plsc.VectorSubcoreMesh(core_axis_name="c", subcore_axis_name="s")
    params = pltpu.CompilerParams(needs_layout_passes=False)
    deg_k = functools.partial(
        pl.kernel,
        out_type=jax.ShapeDtypeStruct((TT * NWORK * N,), jnp.float32),
        mesh=mesh,
        compiler_params=params,
        scratch_types=[
            pltpu.VMEM((N,), jnp.float32),
            pltpu.VMEM((DEG_CH,), jnp.int32),
            pltpu.VMEM((DEG_CH,), jnp.float32),
        ],
    )(_sc_deg_body)
    agg_k = functools.partial(
        pl.kernel,
        out_type=jax.ShapeDtypeStruct((TT, 2, NP, D), jnp.float32),
        mesh=mesh,
        compiler_params=params,
        scratch_types=(
            [pltpu.VMEM((N,), jnp.float32)]
            + [pltpu.VMEM((AGG_CH,), jnp.int32) for _ in range(3)]
            + [pltpu.VMEM((AGG_CH,), jnp.int32) for _ in range(3)]
            + [pltpu.VMEM((AGG_CH,), jnp.float32) for _ in range(3)]
            + [pltpu.VMEM((AGG_CH,), jnp.int32) for _ in range(3)]
            + [pltpu.VMEM((AGG_CH,), jnp.int32) for _ in range(3)]
            + [pltpu.VMEM((AGG_CH,), jnp.float32) for _ in range(3)]
            + [pltpu.VMEM((AGG_CH, D), jnp.float32) for _ in range(3)]
            + [pltpu.VMEM_SHARED((NP, D), jnp.float32)]
            + [pltpu.SemaphoreType.DMA for _ in range(9)]
        ),
    )(_sc_agg_body)
    return deg_k, agg_k


def _sc_deg_body(ei_hbm, ew_hbm, z_hbm, out_hbm, deg_v, idx_v, ew_v):
    cid = lax.axis_index("c")
    sid = lax.axis_index("s")
    wid = sid * 2 + cid
    for t in range(TT):
        def zero_body(i, c):
            deg_v[pl.ds(i * 16, 16)] = jnp.zeros((16,), jnp.float32)
            return c
        lax.fori_loop(0, N // 16, zero_body, 0)
        dbase = (2 * t + 1) * E + wid * EPW
        ebase = t * E + wid * EPW
        for g in range(EPW // DEG_CH):
            pltpu.sync_copy(ei_hbm.at[pl.ds(dbase + g * DEG_CH, DEG_CH)],
                            idx_v)
            pltpu.sync_copy(ew_hbm.at[pl.ds(ebase + g * DEG_CH, DEG_CH)],
                            ew_v)

            def acc_body(j, c):
                for u in range(5):
                    sl = pl.ds(j * 80 + u * 16, 16)
                    ii = idx_v[sl]
                    w = ew_v[sl]
                    sig = 1.0 / (1.0 + jnp.exp(-w))
                    plsc.addupdate_scatter(deg_v, [ii], sig)
                return c
            lax.fori_loop(0, DEG_CH // 80, acc_body, 0)
        pltpu.sync_copy(deg_v, out_hbm.at[pl.ds((t * NWORK + wid) * N, N)])


def _sc_agg_body(ei_hbm, ew_hbm, dinv_hbm, xw_hbm, zer_hbm,
                 out_hbm, dinv_v, *sc):
    cid = lax.axis_index("c")
    sid = lax.axis_index("s")
    NCH = EPW // AGG_CH
    sraw, draw, eraw = sc[0:3], sc[3:6], sc[6:9]
    gsrc, gdst, nrm = sc[9:12], sc[12:15], sc[15:18]
    rows = sc[18:21]
    acc_sh = sc[21]
    sidx, sgat, ssc = sc[22:25], sc[25:28], sc[28:31]

    tile0 = sid * RPTP

    def tbody(t, tc):
        ebase = cid * (E // 2) + sid * EPW

        def fire_idx(g, b):
            soff = 2 * t * E + ebase + g * AGG_CH
            woff = t * E + ebase + g * AGG_CH
            pltpu.async_copy(ei_hbm.at[pl.ds(soff, AGG_CH)], sraw[b], sidx[b])
            pltpu.async_copy(ei_hbm.at[pl.ds(soff + E, AGG_CH)], draw[b],
                             sidx[b])
            pltpu.async_copy(ew_hbm.at[pl.ds(woff, AGG_CH)], eraw[b], sidx[b])

        def wait_idx(g, b):
            soff = 2 * t * E + ebase + g * AGG_CH
            woff = t * E + ebase + g * AGG_CH
            pltpu.make_async_copy(ei_hbm.at[pl.ds(soff, AGG_CH)], sraw[b],
                                  sidx[b]).wait()
            pltpu.make_async_copy(ei_hbm.at[pl.ds(soff + E, AGG_CH)], draw[b],
                                  sidx[b]).wait()
            pltpu.make_async_copy(ew_hbm.at[pl.ds(woff, AGG_CH)], eraw[b],
                                  sidx[b]).wait()

        def prep(b):
            for j in range(AGG_CH // 16):
                sl = pl.ds(j * 16, 16)
                si = sraw[b][sl]
                w = eraw[b][sl]
                sig = 1.0 / (1.0 + jnp.exp(-w))
                dv = plsc.load_gather(dinv_v, [si])
                nrm[b][sl] = sig * dv
                gsrc[b][sl] = si
                gdst[b][sl] = draw[b][sl]
            pltpu.async_copy(xw_hbm.at[t].at[gsrc[b]], rows[b], sgat[b])

        def wait_gather(b):
            pltpu.make_async_copy(
                xw_hbm.at[t].at[gsrc[b]], rows[b], sgat[b]).wait()

        def scale(b):
            def sg(jg, c):
                nj16 = nrm[b][pl.ds(jg * 16, 16)]
                for lane in range(16):
                    nj = nj16[lane]
                    row = jg * 16 + lane
                    for cc in range(8):
                        csl = pl.ds(cc * 16, 16)
                        rows[b][row, csl] = rows[b][row, csl] * nj
                return c
            lax.fori_loop(0, AGG_CH // 16, sg, 0)

        def fire_scatter(b):
            pltpu.async_copy(rows[b], acc_sh.at[gdst[b]], ssc[b], add=True)

        def wait_scatter(b):
            pltpu.make_async_copy(rows[b], acc_sh.at[gdst[b]], ssc[b]).wait()

        pltpu.sync_copy(zer_hbm, acc_sh.at[pl.ds(tile0, RPTP)])
        pltpu.sync_copy(dinv_hbm.at[pl.ds(t * N, N)], dinv_v)
        plsc.subcore_barrier()

        fire_idx(0, 0)
        wait_idx(0, 0)
        prep(0)
        fire_idx(1, 1)

        def body(i, c):
            for k in range(3):
                g = 3 * i + k
                b, bn, bnn = k, (k + 1) % 3, (k + 2) % 3
                if k < 2:
                    @pl.when(i > 0)
                    def _():
                        wait_scatter(bn)
                else:
                    wait_scatter(bn)
                wait_idx(g + 1, bn)
                prep(bn)
                fire_idx(g + 2, bnn)
                wait_gather(b)
                scale(b)
                fire_scatter(b)
            return c
        lax.fori_loop(0, (NCH - 2) // 3, body, 0)

        wait_scatter(1)
        wait_idx(NCH - 1, 1)
        prep(1)
        wait_gather(0)
        scale(0)
        fire_scatter(0)
        wait_scatter(2)
        wait_gather(1)
        scale(1)
        fire_scatter(1)
        wait_scatter(0)
        wait_scatter(1)
        plsc.subcore_barrier()
        sl = pl.ds(tile0, RPTP)
        pltpu.sync_copy(acc_sh.at[sl], out_hbm.at[t, cid, sl])
        plsc.subcore_barrier()
        return tc

    lax.fori_loop(0, TT, tbody, 0)


def kernel(X_list, edge_index_list, W_list, Y_hist_list, hidden_in,
           edge_weight_list, W_phi, b_phi, gc_W, gc_b, W_fuse, b_fuse,
           Wih, bih, Whh, bhh, Wd, bd, dw1, db1, dw2, db2):
    ei = edge_index_list.astype(jnp.int32).reshape(TT * 2 * E)
    ew = edge_weight_list.reshape(TT * E)

    sc_deg, sc_agg = _sc_kernels()
    phi, xw = _phi_xw(X_list, W_phi, b_phi.reshape(1, D), gc_W)
    deg_parts = sc_deg(ei, ew,
                       jnp.zeros((N,), jnp.float32)).reshape(TT, NWORK, N)
    dinv3, dd3 = _dinv(deg_parts)
    aggp = sc_agg(ei, ew, dinv3.reshape(TT * N), xw,
                  jnp.zeros((RPTP, D), jnp.float32))
    dinv_col = dinv3.reshape(TT, N, 1)
    dd_col = dd3.reshape(TT, N, 1)

    wfp = W_fuse[:D]
    wfh = W_fuse[D:2 * D]
    wfr = W_fuse[2 * D:]
    wd_pad = jnp.zeros((D, D), jnp.float32).at[:, :GRID + 1].set(Wd)
    bd_pad = jnp.full((1, D), -1e30, jnp.float32).at[0, :GRID + 1].set(bd)
    dw1m = dw1.transpose(2, 0, 1)
    db1t = db1.T
    dw2f = dw2[:, 0, :]
    wihz = Wih[:, :D].T
    wihw = Wih[:, D][None]
    wihy = Wih[:, D + 1][None]
    whht = Whh.T

    h = hidden_in
    gcb3 = gc_b.reshape(TT, 1, D)
    ys, gs = [], []
    for t in range(TT):
        y_t, g_t, h = _stage2(
            t, phi, xw, aggp, dinv_col, dd_col, h, W_list, Y_hist_list,
            gcb3, wfp, wfh, wfr, b_fuse[None], wd_pad, bd_pad, dw1m, db1t,
            dw2f, db2, wihz, wihw, wihy, bih[None], whht, bhh[None])
        ys.append(y_t)
        gs.append(g_t[:, 0])
    return jnp.stack(ys), jnp.stack(gs), h
